# baseline (speedup 1.0000x reference)
import jax
import jax.numpy as jnp
from jax.experimental import pallas as pl
from jax.experimental.pallas import tpu as pltpu

_LB = 1024


def _rows_body(maskp_ref, perm_ref, src_ref, steps_ref, pred_ref, z_ref,
               pi_ref, zi_ref, po_ref, zo_ref, steps_out_ref):
    j = pl.program_id(0)
    t = pl.program_id(1)

    m = maskp_ref[t] != 0
    row = perm_ref[t]
    steps_out_ref[row] = jnp.where(m, jnp.int32(0), steps_ref[row])

    @pl.when(jnp.logical_and(m, t < 2))
    def _():
        po_ref[0] = jnp.broadcast_to(pi_ref[0], po_ref.shape[1:])
        zo_ref[0] = jnp.broadcast_to(zi_ref[0], zo_ref.shape[1:])

    @pl.when(jnp.logical_not(m))
    def _():
        po_ref[...] = pred_ref[...]
        zo_ref[...] = z_ref[...]


def kernel(prediction_y, reasoning_Z, steps, mask, pred_init, Z_init):
    B, L, D = prediction_y.shape
    J = L // _LB
    mask_i = mask.astype(jnp.int32)

    unm = mask_i == 0
    n_masked = jnp.sum(mask_i)
    rank_u = jnp.cumsum(unm.astype(jnp.int32)) - 1
    rank_m = jnp.cumsum(mask_i) - 1
    step = jnp.where(unm, n_masked + rank_u, rank_m)
    perm = jnp.argsort(step).astype(jnp.int32)

    unm_p = jnp.take(unm, perm)
    first_unm_row = perm[jnp.argmax(unm_p)]
    src_row = jnp.where(unm_p, perm, first_unm_row).astype(jnp.int32)
    mask_p = jnp.take(mask_i, perm)

    def in_map(j, t, maskp_ref, perm_ref, src_ref):
        return (src_ref[t], j, 0)

    def out_map(j, t, maskp_ref, perm_ref, src_ref):
        return (perm_ref[t], j, 0)

    def init_map(j, t, maskp_ref, perm_ref, src_ref):
        return (0, 0, 0)

    grid_spec = pltpu.PrefetchScalarGridSpec(
        num_scalar_prefetch=3,
        grid=(J, B),
        in_specs=[
            pl.BlockSpec(memory_space=pltpu.SMEM),
            pl.BlockSpec((1, _LB, D), in_map),
            pl.BlockSpec((1, _LB, D), in_map),
            pl.BlockSpec((1, 1, D), init_map),
            pl.BlockSpec((1, 1, D), init_map),
        ],
        out_specs=[
            pl.BlockSpec((1, _LB, D), out_map),
            pl.BlockSpec((1, _LB, D), out_map),
            pl.BlockSpec(memory_space=pltpu.SMEM),
        ],
    )
    pred_out, Z_out, steps_out = pl.pallas_call(
        _rows_body,
        grid_spec=grid_spec,
        out_shape=[
            jax.ShapeDtypeStruct((B, L, D), jnp.float32),
            jax.ShapeDtypeStruct((B, L, D), jnp.float32),
            jax.ShapeDtypeStruct((B,), jnp.int32),
        ],
    )(mask_p, perm, src_row, steps, prediction_y, reasoning_Z, pred_init, Z_init)
    return (pred_out, Z_out, steps_out)
